# Initial kernel scaffold; baseline (speedup 1.0000x reference)
import jax
import jax.numpy as jnp
from jax.experimental import pallas as pl


def kernel(x, edge_index, edge_weights, weight, bias, bn_gamma, bn_beta):
    raise NotImplementedError("write your pallas kernel here")


# speedup vs baseline: 123.3637x; 123.3637x over previous
import jax
import jax.numpy as jnp
from jax import lax
from jax.experimental import pallas as pl
from jax.experimental.pallas import tpu as pltpu
from jax.experimental.pallas import tpu_sc as plsc

_N = 100000
_E = 3200000
_ROWW = 128
_RPT = 784
_EPAD = 32 * _RPT * _ROWW
_CH = 16
_NCHUNK = _RPT // _CH
_WIN = 10000
_SL = 6256
_SL_LAST = _N - 15 * _SL


def _spmv_body(curA, curB, src2d, dst2d, w2d, outA, outB,
               repl, tmp, sbuf, dbuf, wbuf, vbuf, acc_sh, sem):
    c = lax.axis_index("c")
    s = lax.axis_index("s")
    wid = c * 16 + s

    def _zero(t, carry):
        tmp[pl.ds(t * 16, 16)] = jnp.zeros((16,), jnp.float32)
        return carry
    lax.fori_loop(0, _WIN // 16, _zero, 0)

    @pl.when(s < 15)
    def _():
        pltpu.sync_copy(tmp.at[pl.ds(0, _SL)], acc_sh.at[pl.ds(s * _SL, _SL)])

    @pl.when(s == 15)
    def _():
        pltpu.sync_copy(tmp.at[pl.ds(0, _SL_LAST)],
                        acc_sh.at[pl.ds(15 * _SL, _SL_LAST)])

    pltpu.sync_copy(curA, repl)
    for win in range(_N // _WIN):
        pltpu.sync_copy(curB.at[pl.ds(win * _WIN, _WIN)], tmp)

        def _addw(t, carry):
            off = win * _WIN + t * 16
            repl[pl.ds(off, 16)] = repl[pl.ds(off, 16)] + tmp[pl.ds(t * 16, 16)]
            return carry
        lax.fori_loop(0, _WIN // 16, _addw, 0)

    plsc.subcore_barrier()

    row0 = wid * _RPT

    def _chunk(i, carry):
        rbase = row0 + i * _CH
        pltpu.sync_copy(src2d.at[pl.ds(rbase, _CH)], sbuf)
        pltpu.sync_copy(dst2d.at[pl.ds(rbase, _CH)], dbuf)
        pltpu.sync_copy(w2d.at[pl.ds(rbase, _CH)], wbuf)
        handles = []
        for r in range(_CH):
            for j in range(_ROWW // 16):
                idx = sbuf[r, pl.ds(j * 16, 16)]
                vals = plsc.load_gather(repl, [idx])
                vbuf[r, pl.ds(j * 16, 16)] = vals * wbuf[r, pl.ds(j * 16, 16)]
            handles.append(
                pltpu.async_copy(vbuf.at[r], acc_sh.at[dbuf.at[r]], sem,
                                 add=True))
        for h in handles:
            h.wait()
        return carry
    lax.fori_loop(0, _NCHUNK, _chunk, 0)

    plsc.subcore_barrier()

    sl_base = s * _SL

    @pl.when(s < 15)
    def _():
        pltpu.sync_copy(acc_sh.at[pl.ds(sl_base, _SL)], tmp.at[pl.ds(0, _SL)])

        @pl.when(c == 0)
        def _():
            pltpu.sync_copy(tmp.at[pl.ds(0, _SL)], outA.at[pl.ds(sl_base, _SL)])

        @pl.when(c == 1)
        def _():
            pltpu.sync_copy(tmp.at[pl.ds(0, _SL)], outB.at[pl.ds(sl_base, _SL)])

    @pl.when(s == 15)
    def _():
        pltpu.sync_copy(acc_sh.at[pl.ds(15 * _SL, _SL_LAST)],
                        tmp.at[pl.ds(0, _SL_LAST)])

        @pl.when(c == 0)
        def _():
            pltpu.sync_copy(tmp.at[pl.ds(0, _SL_LAST)],
                            outA.at[pl.ds(15 * _SL, _SL_LAST)])

        @pl.when(c == 1)
        def _():
            pltpu.sync_copy(tmp.at[pl.ds(0, _SL_LAST)],
                            outB.at[pl.ds(15 * _SL, _SL_LAST)])


_spmv = pl.kernel(
    _spmv_body,
    out_type=(jax.ShapeDtypeStruct((_N,), jnp.float32),
              jax.ShapeDtypeStruct((_N,), jnp.float32)),
    mesh=plsc.VectorSubcoreMesh(core_axis_name="c", subcore_axis_name="s"),
    scratch_types=[
        pltpu.VMEM((_N,), jnp.float32),
        pltpu.VMEM((_WIN,), jnp.float32),
        pltpu.VMEM((_CH, _ROWW), jnp.int32),
        pltpu.VMEM((_CH, _ROWW), jnp.int32),
        pltpu.VMEM((_CH, _ROWW), jnp.float32),
        pltpu.VMEM((_CH, _ROWW), jnp.float32),
        pltpu.VMEM_SHARED((_N,), jnp.float32),
        pltpu.SemaphoreType.DMA,
    ],
    compiler_params=pltpu.CompilerParams(needs_layout_passes=False),
)


_B = 100
_BB = 10
_ROWS_BLK = _BB * 1000


def _stats_body(f8, x2, h8, h0, b2, mean_out, inv_out, acc1, acc2):
    i = pl.program_id(0)
    y = (jax.lax.dot_general(x2[...], h0[...], (((1,), (0,)), ((), ())),
                             preferred_element_type=jnp.float32)
         + jax.lax.dot_general(f8[...], h8[...], (((1,), (0,)), ((), ())),
                               preferred_element_type=jnp.float32)
         + b2[...])
    s1 = jnp.zeros((1000, 64), jnp.float32)
    s2 = jnp.zeros((1000, 64), jnp.float32)
    for b in range(_BB):
        yb = y[b * 1000:(b + 1) * 1000, :]
        s1 = s1 + yb
        s2 = s2 + yb * yb

    @pl.when(i == 0)
    def _():
        acc1[...] = s1
        acc2[...] = s2

    @pl.when(i > 0)
    def _():
        acc1[...] = acc1[...] + s1
        acc2[...] = acc2[...] + s2

    @pl.when(i == _B // _BB - 1)
    def _():
        denom = float(_B * 64)
        m = jnp.sum(acc1[...], axis=1, keepdims=True) / denom
        ey2 = jnp.sum(acc2[...], axis=1, keepdims=True) / denom
        var = ey2 - m * m
        mean_out[...] = m
        inv_out[...] = jax.lax.rsqrt(var + 1e-5)


def _norm_body(f8, x2, h8, h0, b2, g2, be2, mean, inv, out):
    y = (jax.lax.dot_general(x2[...], h0[...], (((1,), (0,)), ((), ())),
                             preferred_element_type=jnp.float32)
         + jax.lax.dot_general(f8[...], h8[...], (((1,), (0,)), ((), ())),
                               preferred_element_type=jnp.float32)
         + b2[...])
    scale = inv[...] * g2[...]
    shift = be2[...] - mean[...] * scale
    scale_r = jnp.concatenate([scale] * _BB, axis=0)
    shift_r = jnp.concatenate([shift] * _BB, axis=0)
    res = y * scale_r + shift_r
    out[...] = res.reshape(_BB, 1000, 64)


def kernel(x, edge_index, edge_weights, weight, bias, bn_gamma, bn_beta):
    src = edge_index[0]
    dst = edge_index[1]
    pad = _EPAD - _E
    pad_dst = (jax.lax.iota(jnp.int32, pad) * 887) % _N
    src_p = jnp.concatenate([src, jnp.zeros((pad,), jnp.int32)]
                            ).reshape(_EPAD // _ROWW, _ROWW)
    dst_p = jnp.concatenate([dst, pad_dst]).reshape(_EPAD // _ROWW, _ROWW)
    w_p = jnp.concatenate([edge_weights, jnp.zeros((pad,), jnp.float32)]
                          ).reshape(_EPAD // _ROWW, _ROWW)
    xf = x.reshape(_N)
    zero_n = jnp.zeros((_N,), jnp.float32)

    a1 = _spmv(xf, zero_n, src_p, dst_p, w_p)
    a2 = _spmv(a1[0], a1[1], src_p, dst_p, w_p)
    a3 = _spmv(a2[0], a2[1], src_p, dst_p, w_p)
    a4 = _spmv(a3[0], a3[1], src_p, dst_p, w_p)

    f8 = jnp.stack([a1[0], a1[1], a2[0], a2[1], a3[0], a3[1], a4[0], a4[1]],
                   axis=1)
    h = jnp.transpose(weight.reshape(64, 5), (1, 0))
    h8 = jnp.repeat(h[1:5], 2, axis=0)
    h0 = h[0:1]
    b2 = bias.reshape(1, 64)
    g2 = bn_gamma.reshape(1000, 1)
    be2 = bn_beta.reshape(1000, 1)
    x2 = x.reshape(_N, 1)

    grid = (_B // _BB,)
    mean, inv = pl.pallas_call(
        _stats_body,
        grid=grid,
        in_specs=[
            pl.BlockSpec((_ROWS_BLK, 8), lambda i: (i, 0)),
            pl.BlockSpec((_ROWS_BLK, 1), lambda i: (i, 0)),
            pl.BlockSpec((8, 64), lambda i: (0, 0)),
            pl.BlockSpec((1, 64), lambda i: (0, 0)),
            pl.BlockSpec((1, 64), lambda i: (0, 0)),
        ],
        out_specs=[
            pl.BlockSpec((1000, 1), lambda i: (0, 0)),
            pl.BlockSpec((1000, 1), lambda i: (0, 0)),
        ],
        out_shape=[
            jax.ShapeDtypeStruct((1000, 1), jnp.float32),
            jax.ShapeDtypeStruct((1000, 1), jnp.float32),
        ],
        scratch_shapes=[
            pltpu.VMEM((1000, 64), jnp.float32),
            pltpu.VMEM((1000, 64), jnp.float32),
        ],
    )(f8, x2, h8, h0, b2)

    out = pl.pallas_call(
        _norm_body,
        grid=grid,
        in_specs=[
            pl.BlockSpec((_ROWS_BLK, 8), lambda i: (i, 0)),
            pl.BlockSpec((_ROWS_BLK, 1), lambda i: (i, 0)),
            pl.BlockSpec((8, 64), lambda i: (0, 0)),
            pl.BlockSpec((1, 64), lambda i: (0, 0)),
            pl.BlockSpec((1, 64), lambda i: (0, 0)),
            pl.BlockSpec((1000, 1), lambda i: (0, 0)),
            pl.BlockSpec((1000, 1), lambda i: (0, 0)),
            pl.BlockSpec((1000, 1), lambda i: (0, 0)),
            pl.BlockSpec((1000, 1), lambda i: (0, 0)),
        ],
        out_specs=pl.BlockSpec((_BB, 1000, 64), lambda i: (i, 0, 0)),
        out_shape=jax.ShapeDtypeStruct((_B, 1000, 64), jnp.float32),
    )(f8, x2, h8, h0, b2, g2, be2, mean, inv)
    return out


# speedup vs baseline: 170.4008x; 1.3813x over previous
import jax
import jax.numpy as jnp
from jax import lax
from jax.experimental import pallas as pl
from jax.experimental.pallas import tpu as pltpu
from jax.experimental.pallas import tpu_sc as plsc

_N = 100000
_E = 3200000
_ROWW = 128
_RPT = 800
_EPAD = 32 * _RPT * _ROWW
_CH = 16
_NCHUNK = _RPT // _CH
_WIN = 10000
_SL = 6256
_SL_LAST = _N - 15 * _SL


def _spmv_body(curA, curB, zeros_h, src2d, dst2d, w2d, outA, outB,
               repl, tmp, sbuf, dbuf, wbuf, vbuf, acc_sh, sem_in, sem_sc):
    c = lax.axis_index("c")
    s = lax.axis_index("s")
    wid = c * 16 + s

    pltpu.sync_copy(zeros_h.at[pl.ds(0, _WIN)], tmp)

    @pl.when(s < 15)
    def _():
        pltpu.sync_copy(tmp.at[pl.ds(0, _SL)], acc_sh.at[pl.ds(s * _SL, _SL)])

    @pl.when(s == 15)
    def _():
        pltpu.sync_copy(tmp.at[pl.ds(0, _SL_LAST)],
                        acc_sh.at[pl.ds(15 * _SL, _SL_LAST)])

    pltpu.sync_copy(curA, repl)
    for win in range(_N // _WIN):
        pltpu.sync_copy(curB.at[pl.ds(win * _WIN, _WIN)], tmp)

        def _addw(t, carry):
            off = t * 80
            for u in range(5):
                o = off + u * 16
                repl[pl.ds(win * _WIN + o, 16)] = (
                    repl[pl.ds(win * _WIN + o, 16)] + tmp[pl.ds(o, 16)])
            return carry
        lax.fori_loop(0, _WIN // 80, _addw, 0)

    plsc.subcore_barrier()

    row0 = wid * _RPT

    def _half(k, b, first):
        if not first:
            pltpu.make_async_copy(w2d.at[pl.ds(0, _CH)], vbuf.at[b],
                                  sem_sc).wait()
        rbase = row0 + k * _CH
        h1 = pltpu.async_copy(src2d.at[pl.ds(rbase, _CH)], sbuf, sem_in)
        h2 = pltpu.async_copy(dst2d.at[pl.ds(rbase, _CH)], dbuf.at[b], sem_in)
        h3 = pltpu.async_copy(w2d.at[pl.ds(rbase, _CH)], wbuf, sem_in)
        h1.wait()
        h2.wait()
        h3.wait()
        for r in range(_CH):
            for j in range(_ROWW // 16):
                idx = sbuf[r, pl.ds(j * 16, 16)]
                vals = plsc.load_gather(repl, [idx])
                vbuf[b, r, pl.ds(j * 16, 16)] = (
                    vals * wbuf[r, pl.ds(j * 16, 16)])
            pltpu.async_copy(vbuf.at[b, r], acc_sh.at[dbuf.at[b, r]], sem_sc,
                             add=True)

    _half(0, 0, True)
    _half(1, 1, True)

    def _pair(i, carry):
        _half(2 * i, 0, False)
        _half(2 * i + 1, 1, False)
        return carry
    lax.fori_loop(1, _NCHUNK // 2, _pair, 0)

    pltpu.make_async_copy(w2d.at[pl.ds(0, _CH)], vbuf.at[0], sem_sc).wait()
    pltpu.make_async_copy(w2d.at[pl.ds(0, _CH)], vbuf.at[1], sem_sc).wait()

    plsc.subcore_barrier()

    sl_base = s * _SL

    @pl.when(s < 15)
    def _():
        pltpu.sync_copy(acc_sh.at[pl.ds(sl_base, _SL)], tmp.at[pl.ds(0, _SL)])

        @pl.when(c == 0)
        def _():
            pltpu.sync_copy(tmp.at[pl.ds(0, _SL)], outA.at[pl.ds(sl_base, _SL)])

        @pl.when(c == 1)
        def _():
            pltpu.sync_copy(tmp.at[pl.ds(0, _SL)], outB.at[pl.ds(sl_base, _SL)])

    @pl.when(s == 15)
    def _():
        pltpu.sync_copy(acc_sh.at[pl.ds(15 * _SL, _SL_LAST)],
                        tmp.at[pl.ds(0, _SL_LAST)])

        @pl.when(c == 0)
        def _():
            pltpu.sync_copy(tmp.at[pl.ds(0, _SL_LAST)],
                            outA.at[pl.ds(15 * _SL, _SL_LAST)])

        @pl.when(c == 1)
        def _():
            pltpu.sync_copy(tmp.at[pl.ds(0, _SL_LAST)],
                            outB.at[pl.ds(15 * _SL, _SL_LAST)])


_spmv = pl.kernel(
    _spmv_body,
    out_type=(jax.ShapeDtypeStruct((_N,), jnp.float32),
              jax.ShapeDtypeStruct((_N,), jnp.float32)),
    mesh=plsc.VectorSubcoreMesh(core_axis_name="c", subcore_axis_name="s"),
    scratch_types=[
        pltpu.VMEM((_N,), jnp.float32),
        pltpu.VMEM((_WIN,), jnp.float32),
        pltpu.VMEM((_CH, _ROWW), jnp.int32),
        pltpu.VMEM((2, _CH, _ROWW), jnp.int32),
        pltpu.VMEM((_CH, _ROWW), jnp.float32),
        pltpu.VMEM((2, _CH, _ROWW), jnp.float32),
        pltpu.VMEM_SHARED((_N,), jnp.float32),
        pltpu.SemaphoreType.DMA,
        pltpu.SemaphoreType.DMA,
    ],
    compiler_params=pltpu.CompilerParams(needs_layout_passes=False),
)


_B = 100
_BB = 10
_ROWS_BLK = _BB * 1000


def _stats_body(f8, x2, h8, h0, b2, mean_out, inv_out, acc1, acc2):
    i = pl.program_id(0)
    y = (jax.lax.dot_general(x2[...], h0[...], (((1,), (0,)), ((), ())),
                             preferred_element_type=jnp.float32)
         + jax.lax.dot_general(f8[...], h8[...], (((1,), (0,)), ((), ())),
                               preferred_element_type=jnp.float32)
         + b2[...])
    s1 = jnp.zeros((1000, 64), jnp.float32)
    s2 = jnp.zeros((1000, 64), jnp.float32)
    for b in range(_BB):
        yb = y[b * 1000:(b + 1) * 1000, :]
        s1 = s1 + yb
        s2 = s2 + yb * yb

    @pl.when(i == 0)
    def _():
        acc1[...] = s1
        acc2[...] = s2

    @pl.when(i > 0)
    def _():
        acc1[...] = acc1[...] + s1
        acc2[...] = acc2[...] + s2

    @pl.when(i == _B // _BB - 1)
    def _():
        denom = float(_B * 64)
        m = jnp.sum(acc1[...], axis=1, keepdims=True) / denom
        ey2 = jnp.sum(acc2[...], axis=1, keepdims=True) / denom
        var = ey2 - m * m
        mean_out[...] = m
        inv_out[...] = jax.lax.rsqrt(var + 1e-5)


def _norm_body(f8, x2, h8, h0, b2, g2, be2, mean, inv, out):
    y = (jax.lax.dot_general(x2[...], h0[...], (((1,), (0,)), ((), ())),
                             preferred_element_type=jnp.float32)
         + jax.lax.dot_general(f8[...], h8[...], (((1,), (0,)), ((), ())),
                               preferred_element_type=jnp.float32)
         + b2[...])
    scale = inv[...] * g2[...]
    shift = be2[...] - mean[...] * scale
    scale_r = jnp.concatenate([scale] * _BB, axis=0)
    shift_r = jnp.concatenate([shift] * _BB, axis=0)
    res = y * scale_r + shift_r
    out[...] = res.reshape(_BB, 1000, 64)


def kernel(x, edge_index, edge_weights, weight, bias, bn_gamma, bn_beta):
    src = edge_index[0]
    dst = edge_index[1]
    pad = _EPAD - _E
    pad_dst = (jax.lax.iota(jnp.int32, pad) * 887) % _N
    src_p = jnp.concatenate([src, jnp.zeros((pad,), jnp.int32)]
                            ).reshape(_EPAD // _ROWW, _ROWW)
    dst_p = jnp.concatenate([dst, pad_dst]).reshape(_EPAD // _ROWW, _ROWW)
    w_p = jnp.concatenate([edge_weights, jnp.zeros((pad,), jnp.float32)]
                          ).reshape(_EPAD // _ROWW, _ROWW)
    xf = x.reshape(_N)
    zero_n = jnp.zeros((_N,), jnp.float32)

    a1 = _spmv(xf, zero_n, zero_n, src_p, dst_p, w_p)
    a2 = _spmv(a1[0], a1[1], zero_n, src_p, dst_p, w_p)
    a3 = _spmv(a2[0], a2[1], zero_n, src_p, dst_p, w_p)
    a4 = _spmv(a3[0], a3[1], zero_n, src_p, dst_p, w_p)

    f8 = jnp.stack([a1[0], a1[1], a2[0], a2[1], a3[0], a3[1], a4[0], a4[1]],
                   axis=1)
    h = jnp.transpose(weight.reshape(64, 5), (1, 0))
    h8 = jnp.repeat(h[1:5], 2, axis=0)
    h0 = h[0:1]
    b2 = bias.reshape(1, 64)
    g2 = bn_gamma.reshape(1000, 1)
    be2 = bn_beta.reshape(1000, 1)
    x2 = x.reshape(_N, 1)

    grid = (_B // _BB,)
    mean, inv = pl.pallas_call(
        _stats_body,
        grid=grid,
        in_specs=[
            pl.BlockSpec((_ROWS_BLK, 8), lambda i: (i, 0)),
            pl.BlockSpec((_ROWS_BLK, 1), lambda i: (i, 0)),
            pl.BlockSpec((8, 64), lambda i: (0, 0)),
            pl.BlockSpec((1, 64), lambda i: (0, 0)),
            pl.BlockSpec((1, 64), lambda i: (0, 0)),
        ],
        out_specs=[
            pl.BlockSpec((1000, 1), lambda i: (0, 0)),
            pl.BlockSpec((1000, 1), lambda i: (0, 0)),
        ],
        out_shape=[
            jax.ShapeDtypeStruct((1000, 1), jnp.float32),
            jax.ShapeDtypeStruct((1000, 1), jnp.float32),
        ],
        scratch_shapes=[
            pltpu.VMEM((1000, 64), jnp.float32),
            pltpu.VMEM((1000, 64), jnp.float32),
        ],
    )(f8, x2, h8, h0, b2)

    out = pl.pallas_call(
        _norm_body,
        grid=grid,
        in_specs=[
            pl.BlockSpec((_ROWS_BLK, 8), lambda i: (i, 0)),
            pl.BlockSpec((_ROWS_BLK, 1), lambda i: (i, 0)),
            pl.BlockSpec((8, 64), lambda i: (0, 0)),
            pl.BlockSpec((1, 64), lambda i: (0, 0)),
            pl.BlockSpec((1, 64), lambda i: (0, 0)),
            pl.BlockSpec((1000, 1), lambda i: (0, 0)),
            pl.BlockSpec((1000, 1), lambda i: (0, 0)),
            pl.BlockSpec((1000, 1), lambda i: (0, 0)),
            pl.BlockSpec((1000, 1), lambda i: (0, 0)),
        ],
        out_specs=pl.BlockSpec((_BB, 1000, 64), lambda i: (i, 0, 0)),
        out_shape=jax.ShapeDtypeStruct((_B, 1000, 64), jnp.float32),
    )(f8, x2, h8, h0, b2, g2, be2, mean, inv)
    return out


# speedup vs baseline: 182.8732x; 1.0732x over previous
import jax
import jax.numpy as jnp
from jax import lax
from jax.experimental import pallas as pl
from jax.experimental.pallas import tpu as pltpu
from jax.experimental.pallas import tpu_sc as plsc

_N = 100000
_E = 3200000
_ROWW = 128
_RPT = 816
_EPAD = 32 * _RPT * _ROWW
_CH = 8
_NCHUNK = _RPT // _CH
_WIN = 4000
_SL = 6256
_SL_LAST = _N - 15 * _SL


def _spmv_body(curA, curB, zeros_h, src2d, dst2d, w2d, outA, outB,
               repl, tmp, sbuf, dbuf, wbuf, vbuf, acc_sh, sem_in, sem_sc):
    c = lax.axis_index("c")
    s = lax.axis_index("s")
    wid = c * 16 + s

    row0 = wid * _RPT

    def _stage(k, set_):
        rbase = row0 + k * _CH
        pltpu.async_copy(src2d.at[pl.ds(rbase, _CH)], sbuf.at[set_], sem_in)
        pltpu.async_copy(dst2d.at[pl.ds(rbase, _CH)], dbuf.at[set_], sem_in)
        pltpu.async_copy(w2d.at[pl.ds(rbase, _CH)], wbuf.at[set_], sem_in)

    _stage(0, 0)

    pltpu.sync_copy(zeros_h.at[pl.ds(0, _WIN)], tmp)

    @pl.when(s < 15)
    def _():
        pltpu.sync_copy(tmp.at[pl.ds(0, _WIN)], acc_sh.at[pl.ds(s * _SL, _WIN)])
        pltpu.sync_copy(tmp.at[pl.ds(0, _SL - _WIN)],
                        acc_sh.at[pl.ds(s * _SL + _WIN, _SL - _WIN)])

    @pl.when(s == 15)
    def _():
        pltpu.sync_copy(tmp.at[pl.ds(0, _WIN)],
                        acc_sh.at[pl.ds(15 * _SL, _WIN)])
        pltpu.sync_copy(tmp.at[pl.ds(0, _SL_LAST - _WIN)],
                        acc_sh.at[pl.ds(15 * _SL + _WIN, _SL_LAST - _WIN)])

    pltpu.sync_copy(curA, repl)
    for win in range(_N // _WIN):
        pltpu.sync_copy(curB.at[pl.ds(win * _WIN, _WIN)], tmp)

        def _addw(t, carry):
            off = t * 80
            for u in range(5):
                o = off + u * 16
                repl[pl.ds(win * _WIN + o, 16)] = (
                    repl[pl.ds(win * _WIN + o, 16)] + tmp[pl.ds(o, 16)])
            return carry
        lax.fori_loop(0, _WIN // 80, _addw, 0)

    plsc.subcore_barrier()

    def _drain1():
        pltpu.make_async_copy(w2d.at[pl.ds(0, _CH)], vbuf.at[0],
                              sem_sc).wait()

    def _wait_stage(set_):
        h = pltpu.make_async_copy(src2d.at[pl.ds(0, _CH)], sbuf.at[set_],
                                  sem_in)
        h.wait()
        h.wait()
        h.wait()

    def _body(k, set_, drain, stage_next):
        if drain:
            _drain1()
        if stage_next:
            _stage(k + 1, (set_ + 1) % 3)
        _wait_stage(set_)
        for r in range(_CH):
            for j in range(_ROWW // 16):
                idx = sbuf[set_, r, pl.ds(j * 16, 16)]
                vals = plsc.load_gather(repl, [idx])
                vbuf[set_, r, pl.ds(j * 16, 16)] = (
                    vals * wbuf[set_, r, pl.ds(j * 16, 16)])
            pltpu.async_copy(vbuf.at[set_, r], acc_sh.at[dbuf.at[set_, r]],
                             sem_sc, add=True)

    _body(0, 0, False, True)
    _body(1, 1, False, True)

    def _triple(i, carry):
        k = 3 * i + 2
        _body(k, 2, True, True)
        _body(k + 1, 0, True, True)
        _body(k + 2, 1, True, True)
        return carry
    lax.fori_loop(0, (_NCHUNK - 3) // 3, _triple, 0)

    _body(_NCHUNK - 1, 2, True, False)
    _drain1()
    _drain1()

    plsc.subcore_barrier()

    sl_base = s * _SL

    @pl.when(s < 15)
    def _():
        pltpu.sync_copy(acc_sh.at[pl.ds(sl_base, _SL)], tmp.at[pl.ds(0, _SL)])

        @pl.when(c == 0)
        def _():
            pltpu.sync_copy(tmp.at[pl.ds(0, _SL)], outA.at[pl.ds(sl_base, _SL)])

        @pl.when(c == 1)
        def _():
            pltpu.sync_copy(tmp.at[pl.ds(0, _SL)], outB.at[pl.ds(sl_base, _SL)])

    @pl.when(s == 15)
    def _():
        pltpu.sync_copy(acc_sh.at[pl.ds(15 * _SL, _SL_LAST)],
                        tmp.at[pl.ds(0, _SL_LAST)])

        @pl.when(c == 0)
        def _():
            pltpu.sync_copy(tmp.at[pl.ds(0, _SL_LAST)],
                            outA.at[pl.ds(15 * _SL, _SL_LAST)])

        @pl.when(c == 1)
        def _():
            pltpu.sync_copy(tmp.at[pl.ds(0, _SL_LAST)],
                            outB.at[pl.ds(15 * _SL, _SL_LAST)])


_spmv = pl.kernel(
    _spmv_body,
    out_type=(jax.ShapeDtypeStruct((_N,), jnp.float32),
              jax.ShapeDtypeStruct((_N,), jnp.float32)),
    mesh=plsc.VectorSubcoreMesh(core_axis_name="c", subcore_axis_name="s"),
    scratch_types=[
        pltpu.VMEM((_N,), jnp.float32),
        pltpu.VMEM((_WIN,), jnp.float32),
        pltpu.VMEM((3, _CH, _ROWW), jnp.int32),
        pltpu.VMEM((3, _CH, _ROWW), jnp.int32),
        pltpu.VMEM((3, _CH, _ROWW), jnp.float32),
        pltpu.VMEM((3, _CH, _ROWW), jnp.float32),
        pltpu.VMEM_SHARED((_N,), jnp.float32),
        pltpu.SemaphoreType.DMA,
        pltpu.SemaphoreType.DMA,
    ],
    compiler_params=pltpu.CompilerParams(needs_layout_passes=False),
)


_B = 100
_BB = 10
_ROWS_BLK = _BB * 1000


def _stats_body(f8, x2, h8, h0, b2, mean_out, inv_out, acc1, acc2):
    i = pl.program_id(0)
    y = (jax.lax.dot_general(x2[...], h0[...], (((1,), (0,)), ((), ())),
                             preferred_element_type=jnp.float32)
         + jax.lax.dot_general(f8[...], h8[...], (((1,), (0,)), ((), ())),
                               preferred_element_type=jnp.float32)
         + b2[...])
    s1 = jnp.zeros((1000, 64), jnp.float32)
    s2 = jnp.zeros((1000, 64), jnp.float32)
    for b in range(_BB):
        yb = y[b * 1000:(b + 1) * 1000, :]
        s1 = s1 + yb
        s2 = s2 + yb * yb

    @pl.when(i == 0)
    def _():
        acc1[...] = s1
        acc2[...] = s2

    @pl.when(i > 0)
    def _():
        acc1[...] = acc1[...] + s1
        acc2[...] = acc2[...] + s2

    @pl.when(i == _B // _BB - 1)
    def _():
        denom = float(_B * 64)
        m = jnp.sum(acc1[...], axis=1, keepdims=True) / denom
        ey2 = jnp.sum(acc2[...], axis=1, keepdims=True) / denom
        var = ey2 - m * m
        mean_out[...] = m
        inv_out[...] = jax.lax.rsqrt(var + 1e-5)


def _norm_body(f8, x2, h8, h0, b2, g2, be2, mean, inv, out):
    y = (jax.lax.dot_general(x2[...], h0[...], (((1,), (0,)), ((), ())),
                             preferred_element_type=jnp.float32)
         + jax.lax.dot_general(f8[...], h8[...], (((1,), (0,)), ((), ())),
                               preferred_element_type=jnp.float32)
         + b2[...])
    scale = inv[...] * g2[...]
    shift = be2[...] - mean[...] * scale
    scale_r = jnp.concatenate([scale] * _BB, axis=0)
    shift_r = jnp.concatenate([shift] * _BB, axis=0)
    res = y * scale_r + shift_r
    out[...] = res.reshape(_BB, 1000, 64)


def kernel(x, edge_index, edge_weights, weight, bias, bn_gamma, bn_beta):
    src = edge_index[0]
    dst = edge_index[1]
    pad = _EPAD - _E
    pad_dst = (jax.lax.iota(jnp.int32, pad) * 887) % _N
    src_p = jnp.concatenate([src, jnp.zeros((pad,), jnp.int32)]
                            ).reshape(_EPAD // _ROWW, _ROWW)
    dst_p = jnp.concatenate([dst, pad_dst]).reshape(_EPAD // _ROWW, _ROWW)
    w_p = jnp.concatenate([edge_weights, jnp.zeros((pad,), jnp.float32)]
                          ).reshape(_EPAD // _ROWW, _ROWW)
    xf = x.reshape(_N)
    zero_n = jnp.zeros((_N,), jnp.float32)

    a1 = _spmv(xf, zero_n, zero_n, src_p, dst_p, w_p)
    a2 = _spmv(a1[0], a1[1], zero_n, src_p, dst_p, w_p)
    a3 = _spmv(a2[0], a2[1], zero_n, src_p, dst_p, w_p)
    a4 = _spmv(a3[0], a3[1], zero_n, src_p, dst_p, w_p)

    f8 = jnp.stack([a1[0], a1[1], a2[0], a2[1], a3[0], a3[1], a4[0], a4[1]],
                   axis=1)
    h = jnp.transpose(weight.reshape(64, 5), (1, 0))
    h8 = jnp.repeat(h[1:5], 2, axis=0)
    h0 = h[0:1]
    b2 = bias.reshape(1, 64)
    g2 = bn_gamma.reshape(1000, 1)
    be2 = bn_beta.reshape(1000, 1)
    x2 = x.reshape(_N, 1)

    grid = (_B // _BB,)
    mean, inv = pl.pallas_call(
        _stats_body,
        grid=grid,
        in_specs=[
            pl.BlockSpec((_ROWS_BLK, 8), lambda i: (i, 0)),
            pl.BlockSpec((_ROWS_BLK, 1), lambda i: (i, 0)),
            pl.BlockSpec((8, 64), lambda i: (0, 0)),
            pl.BlockSpec((1, 64), lambda i: (0, 0)),
            pl.BlockSpec((1, 64), lambda i: (0, 0)),
        ],
        out_specs=[
            pl.BlockSpec((1000, 1), lambda i: (0, 0)),
            pl.BlockSpec((1000, 1), lambda i: (0, 0)),
        ],
        out_shape=[
            jax.ShapeDtypeStruct((1000, 1), jnp.float32),
            jax.ShapeDtypeStruct((1000, 1), jnp.float32),
        ],
        scratch_shapes=[
            pltpu.VMEM((1000, 64), jnp.float32),
            pltpu.VMEM((1000, 64), jnp.float32),
        ],
    )(f8, x2, h8, h0, b2)

    out = pl.pallas_call(
        _norm_body,
        grid=grid,
        in_specs=[
            pl.BlockSpec((_ROWS_BLK, 8), lambda i: (i, 0)),
            pl.BlockSpec((_ROWS_BLK, 1), lambda i: (i, 0)),
            pl.BlockSpec((8, 64), lambda i: (0, 0)),
            pl.BlockSpec((1, 64), lambda i: (0, 0)),
            pl.BlockSpec((1, 64), lambda i: (0, 0)),
            pl.BlockSpec((1000, 1), lambda i: (0, 0)),
            pl.BlockSpec((1000, 1), lambda i: (0, 0)),
            pl.BlockSpec((1000, 1), lambda i: (0, 0)),
            pl.BlockSpec((1000, 1), lambda i: (0, 0)),
        ],
        out_specs=pl.BlockSpec((_BB, 1000, 64), lambda i: (i, 0, 0)),
        out_shape=jax.ShapeDtypeStruct((_B, 1000, 64), jnp.float32),
    )(f8, x2, h8, h0, b2, g2, be2, mean, inv)
    return out


# speedup vs baseline: 192.5027x; 1.0527x over previous
import jax
import jax.numpy as jnp
from jax import lax
from jax.experimental import pallas as pl
from jax.experimental.pallas import tpu as pltpu
from jax.experimental.pallas import tpu_sc as plsc

_N = 100000
_E = 3200000
_ROWW = 128
_ROWS = _E // _ROWW
_CH = 8
_CBIG, _CSML = 100, 97
_NBIG = 7
_WIN = 4000
_SL = 6256
_SL_LAST = _N - 15 * _SL


def _spmv_body(curA, curB, zeros_h, src2d, dst2d, w2d, outA, outB,
               repl, tmp, sbuf, dbuf, wbuf, vbuf, acc_sh, sem_in, sem_sc):
    c = lax.axis_index("c")
    s = lax.axis_index("s")
    wid = c * 16 + s

    nch = jnp.where(wid < _NBIG, _CBIG, _CSML)
    row0 = jnp.where(wid < _NBIG, wid * (_CBIG * _CH),
                     _NBIG * (_CBIG * _CH) + (wid - _NBIG) * (_CSML * _CH))

    def _stage(k, set_):
        rbase = row0 + k * _CH
        pltpu.async_copy(src2d.at[pl.ds(rbase, _CH)], sbuf.at[set_], sem_in)
        pltpu.async_copy(dst2d.at[pl.ds(rbase, _CH)], dbuf.at[set_], sem_in)
        pltpu.async_copy(w2d.at[pl.ds(rbase, _CH)], wbuf.at[set_], sem_in)

    _stage(0, 0)

    pltpu.sync_copy(zeros_h.at[pl.ds(0, _WIN)], tmp)

    @pl.when(s < 15)
    def _():
        pltpu.sync_copy(tmp.at[pl.ds(0, _WIN)], acc_sh.at[pl.ds(s * _SL, _WIN)])
        pltpu.sync_copy(tmp.at[pl.ds(0, _SL - _WIN)],
                        acc_sh.at[pl.ds(s * _SL + _WIN, _SL - _WIN)])

    @pl.when(s == 15)
    def _():
        pltpu.sync_copy(tmp.at[pl.ds(0, _WIN)],
                        acc_sh.at[pl.ds(15 * _SL, _WIN)])
        pltpu.sync_copy(tmp.at[pl.ds(0, _SL_LAST - _WIN)],
                        acc_sh.at[pl.ds(15 * _SL + _WIN, _SL_LAST - _WIN)])

    pltpu.sync_copy(curA, repl)
    for win in range(_N // _WIN):
        pltpu.sync_copy(curB.at[pl.ds(win * _WIN, _WIN)], tmp)

        def _addw(t, carry):
            off = t * 80
            for u in range(5):
                o = off + u * 16
                repl[pl.ds(win * _WIN + o, 16)] = (
                    repl[pl.ds(win * _WIN + o, 16)] + tmp[pl.ds(o, 16)])
            return carry
        lax.fori_loop(0, _WIN // 80, _addw, 0)

    plsc.subcore_barrier()

    def _drain1():
        pltpu.make_async_copy(w2d.at[pl.ds(0, _CH)], vbuf.at[0],
                              sem_sc).wait()

    def _wait_stage(set_):
        h = pltpu.make_async_copy(src2d.at[pl.ds(0, _CH)], sbuf.at[set_],
                                  sem_in)
        h.wait()
        h.wait()
        h.wait()

    def _body(k, set_, drain, stage_next):
        if drain:
            _drain1()
        if stage_next:
            _stage(k + 1, (set_ + 1) % 3)
        _wait_stage(set_)
        for r in range(_CH):
            for j in range(_ROWW // 16):
                idx = sbuf[set_, r, pl.ds(j * 16, 16)]
                vals = plsc.load_gather(repl, [idx])
                vbuf[set_, r, pl.ds(j * 16, 16)] = (
                    vals * wbuf[set_, r, pl.ds(j * 16, 16)])
            pltpu.async_copy(vbuf.at[set_, r], acc_sh.at[dbuf.at[set_, r]],
                             sem_sc, add=True)

    _body(0, 0, False, True)
    _body(1, 1, False, True)

    def _triple(i, carry):
        k = 3 * i + 2
        _body(k, 2, True, True)
        _body(k + 1, 0, True, True)
        _body(k + 2, 1, True, True)
        return carry
    lax.fori_loop(0, (nch - 4) // 3, _triple, 0)

    _body(nch - 2, 2, True, True)
    _body(nch - 1, 0, True, False)
    _drain1()
    _drain1()

    plsc.subcore_barrier()

    sl_base = s * _SL

    @pl.when(s < 15)
    def _():
        pltpu.sync_copy(acc_sh.at[pl.ds(sl_base, _SL)], tmp.at[pl.ds(0, _SL)])

        @pl.when(c == 0)
        def _():
            pltpu.sync_copy(tmp.at[pl.ds(0, _SL)], outA.at[pl.ds(sl_base, _SL)])

        @pl.when(c == 1)
        def _():
            pltpu.sync_copy(tmp.at[pl.ds(0, _SL)], outB.at[pl.ds(sl_base, _SL)])

    @pl.when(s == 15)
    def _():
        pltpu.sync_copy(acc_sh.at[pl.ds(15 * _SL, _SL_LAST)],
                        tmp.at[pl.ds(0, _SL_LAST)])

        @pl.when(c == 0)
        def _():
            pltpu.sync_copy(tmp.at[pl.ds(0, _SL_LAST)],
                            outA.at[pl.ds(15 * _SL, _SL_LAST)])

        @pl.when(c == 1)
        def _():
            pltpu.sync_copy(tmp.at[pl.ds(0, _SL_LAST)],
                            outB.at[pl.ds(15 * _SL, _SL_LAST)])


_spmv = pl.kernel(
    _spmv_body,
    out_type=(jax.ShapeDtypeStruct((_N,), jnp.float32),
              jax.ShapeDtypeStruct((_N,), jnp.float32)),
    mesh=plsc.VectorSubcoreMesh(core_axis_name="c", subcore_axis_name="s"),
    scratch_types=[
        pltpu.VMEM((_N,), jnp.float32),
        pltpu.VMEM((_WIN,), jnp.float32),
        pltpu.VMEM((3, _CH, _ROWW), jnp.int32),
        pltpu.VMEM((3, _CH, _ROWW), jnp.int32),
        pltpu.VMEM((3, _CH, _ROWW), jnp.float32),
        pltpu.VMEM((3, _CH, _ROWW), jnp.float32),
        pltpu.VMEM_SHARED((_N,), jnp.float32),
        pltpu.SemaphoreType.DMA,
        pltpu.SemaphoreType.DMA,
    ],
    compiler_params=pltpu.CompilerParams(needs_layout_passes=False),
)


_B = 100
_BB = 10
_ROWS_BLK = _BB * 1000


def _stats_body(f8, x2, h8, h0, b2, mean_out, inv_out, acc1, acc2):
    i = pl.program_id(0)
    y = (jax.lax.dot_general(x2[...], h0[...], (((1,), (0,)), ((), ())),
                             preferred_element_type=jnp.float32)
         + jax.lax.dot_general(f8[...], h8[...], (((1,), (0,)), ((), ())),
                               preferred_element_type=jnp.float32)
         + b2[...])
    s1 = jnp.zeros((1000, 64), jnp.float32)
    s2 = jnp.zeros((1000, 64), jnp.float32)
    for b in range(_BB):
        yb = y[b * 1000:(b + 1) * 1000, :]
        s1 = s1 + yb
        s2 = s2 + yb * yb

    @pl.when(i == 0)
    def _():
        acc1[...] = s1
        acc2[...] = s2

    @pl.when(i > 0)
    def _():
        acc1[...] = acc1[...] + s1
        acc2[...] = acc2[...] + s2

    @pl.when(i == _B // _BB - 1)
    def _():
        denom = float(_B * 64)
        m = jnp.sum(acc1[...], axis=1, keepdims=True) / denom
        ey2 = jnp.sum(acc2[...], axis=1, keepdims=True) / denom
        var = ey2 - m * m
        mean_out[...] = m
        inv_out[...] = jax.lax.rsqrt(var + 1e-5)


def _norm_body(f8, x2, h8, h0, b2, g2, be2, mean, inv, out):
    y = (jax.lax.dot_general(x2[...], h0[...], (((1,), (0,)), ((), ())),
                             preferred_element_type=jnp.float32)
         + jax.lax.dot_general(f8[...], h8[...], (((1,), (0,)), ((), ())),
                               preferred_element_type=jnp.float32)
         + b2[...])
    scale = inv[...] * g2[...]
    shift = be2[...] - mean[...] * scale
    scale_r = jnp.concatenate([scale] * _BB, axis=0)
    shift_r = jnp.concatenate([shift] * _BB, axis=0)
    res = y * scale_r + shift_r
    out[...] = res.reshape(_BB, 1000, 64)


def kernel(x, edge_index, edge_weights, weight, bias, bn_gamma, bn_beta):
    src_p = edge_index[0].reshape(_ROWS, _ROWW)
    dst_p = edge_index[1].reshape(_ROWS, _ROWW)
    w_p = edge_weights.reshape(_ROWS, _ROWW)
    xf = x.reshape(_N)
    zero_n = jnp.zeros((_N,), jnp.float32)

    a1 = _spmv(xf, zero_n, zero_n, src_p, dst_p, w_p)
    a2 = _spmv(a1[0], a1[1], zero_n, src_p, dst_p, w_p)
    a3 = _spmv(a2[0], a2[1], zero_n, src_p, dst_p, w_p)
    a4 = _spmv(a3[0], a3[1], zero_n, src_p, dst_p, w_p)

    f8 = jnp.stack([a1[0], a1[1], a2[0], a2[1], a3[0], a3[1], a4[0], a4[1]],
                   axis=1)
    h = jnp.transpose(weight.reshape(64, 5), (1, 0))
    h8 = jnp.repeat(h[1:5], 2, axis=0)
    h0 = h[0:1]
    b2 = bias.reshape(1, 64)
    g2 = bn_gamma.reshape(1000, 1)
    be2 = bn_beta.reshape(1000, 1)
    x2 = x.reshape(_N, 1)

    grid = (_B // _BB,)
    mean, inv = pl.pallas_call(
        _stats_body,
        grid=grid,
        in_specs=[
            pl.BlockSpec((_ROWS_BLK, 8), lambda i: (i, 0)),
            pl.BlockSpec((_ROWS_BLK, 1), lambda i: (i, 0)),
            pl.BlockSpec((8, 64), lambda i: (0, 0)),
            pl.BlockSpec((1, 64), lambda i: (0, 0)),
            pl.BlockSpec((1, 64), lambda i: (0, 0)),
        ],
        out_specs=[
            pl.BlockSpec((1000, 1), lambda i: (0, 0)),
            pl.BlockSpec((1000, 1), lambda i: (0, 0)),
        ],
        out_shape=[
            jax.ShapeDtypeStruct((1000, 1), jnp.float32),
            jax.ShapeDtypeStruct((1000, 1), jnp.float32),
        ],
        scratch_shapes=[
            pltpu.VMEM((1000, 64), jnp.float32),
            pltpu.VMEM((1000, 64), jnp.float32),
        ],
    )(f8, x2, h8, h0, b2)

    out = pl.pallas_call(
        _norm_body,
        grid=grid,
        in_specs=[
            pl.BlockSpec((_ROWS_BLK, 8), lambda i: (i, 0)),
            pl.BlockSpec((_ROWS_BLK, 1), lambda i: (i, 0)),
            pl.BlockSpec((8, 64), lambda i: (0, 0)),
            pl.BlockSpec((1, 64), lambda i: (0, 0)),
            pl.BlockSpec((1, 64), lambda i: (0, 0)),
            pl.BlockSpec((1000, 1), lambda i: (0, 0)),
            pl.BlockSpec((1000, 1), lambda i: (0, 0)),
            pl.BlockSpec((1000, 1), lambda i: (0, 0)),
            pl.BlockSpec((1000, 1), lambda i: (0, 0)),
        ],
        out_specs=pl.BlockSpec((_BB, 1000, 64), lambda i: (i, 0, 0)),
        out_shape=jax.ShapeDtypeStruct((_B, 1000, 64), jnp.float32),
    )(f8, x2, h8, h0, b2, g2, be2, mean, inv)
    return out


# speedup vs baseline: 209.7989x; 1.0898x over previous
import jax
import jax.numpy as jnp
from jax import lax
from jax.experimental import pallas as pl
from jax.experimental.pallas import tpu as pltpu
from jax.experimental.pallas import tpu_sc as plsc

_N = 100000
_E = 3200000
_ROWW = 128
_ROWS = _E // _ROWW
_CH = 8
_CBIG, _CSML = 100, 97
_NBIG = 7
_WIN = 4000
_SL = 6256
_SL_LAST = _N - 15 * _SL
_CURW = 4000


def _spmv4_body(xf, zeros_h, src2d, dst2d, w2d,
                outA, outB, cur1, cur2, cur3,
                repl, tmp, sbuf, dbuf, wbuf, vbuf, acc_sh,
                sem_in, sem_sc, rsem):
    c = lax.axis_index("c")
    s = lax.axis_index("s")
    wid = c * 16 + s

    nch = jnp.where(wid < _NBIG, _CBIG, _CSML)
    row0 = jnp.where(wid < _NBIG, wid * (_CBIG * _CH),
                     _NBIG * (_CBIG * _CH) + (wid - _NBIG) * (_CSML * _CH))

    def _stage(k, set_):
        rbase = row0 + k * _CH
        pltpu.async_copy(src2d.at[pl.ds(rbase, _CH)], sbuf.at[set_], sem_in)
        pltpu.async_copy(dst2d.at[pl.ds(rbase, _CH)], dbuf.at[set_], sem_in)
        pltpu.async_copy(w2d.at[pl.ds(rbase, _CH)], wbuf.at[set_], sem_in)

    def _drain1():
        pltpu.make_async_copy(w2d.at[pl.ds(0, _CH)], vbuf.at[0],
                              sem_sc).wait()

    def _wait_stage(set_):
        h = pltpu.make_async_copy(src2d.at[pl.ds(0, _CH)], sbuf.at[set_],
                                  sem_in)
        h.wait()
        h.wait()
        h.wait()

    def _body(k, set_, drain, stage_next):
        if drain:
            _drain1()
        if stage_next:
            _stage(k + 1, (set_ + 1) % 3)
        _wait_stage(set_)
        for r in range(_CH):
            for j in range(_ROWW // 16):
                idx = sbuf[set_, r, pl.ds(j * 16, 16)]
                vals = plsc.load_gather(repl, [idx])
                vbuf[set_, r, pl.ds(j * 16, 16)] = (
                    vals * wbuf[set_, r, pl.ds(j * 16, 16)])
            pltpu.async_copy(vbuf.at[set_, r], acc_sh.at[dbuf.at[set_, r]],
                             sem_sc, add=True)

    def _acc_to(dst):
        @pl.when(s < 15)
        def _():
            base = s * _SL
            pltpu.sync_copy(acc_sh.at[pl.ds(base, _WIN)],
                            tmp.at[pl.ds(0, _WIN)])
            pltpu.sync_copy(tmp.at[pl.ds(0, _WIN)],
                            dst.at[pl.ds(base, _WIN)])
            pltpu.sync_copy(acc_sh.at[pl.ds(base + _WIN, _SL - _WIN)],
                            tmp.at[pl.ds(0, _SL - _WIN)])
            pltpu.sync_copy(tmp.at[pl.ds(0, _SL - _WIN)],
                            dst.at[pl.ds(base + _WIN, _SL - _WIN)])

        @pl.when(s == 15)
        def _():
            base = 15 * _SL
            pltpu.sync_copy(acc_sh.at[pl.ds(base, _WIN)],
                            tmp.at[pl.ds(0, _WIN)])
            pltpu.sync_copy(tmp.at[pl.ds(0, _WIN)],
                            dst.at[pl.ds(base, _WIN)])
            pltpu.sync_copy(acc_sh.at[pl.ds(base + _WIN, _SL_LAST - _WIN)],
                            tmp.at[pl.ds(0, _SL_LAST - _WIN)])
            pltpu.sync_copy(tmp.at[pl.ds(0, _SL_LAST - _WIN)],
                            dst.at[pl.ds(base + _WIN, _SL_LAST - _WIN)])

    def _round(rnd, carry):
        _stage(0, 0)

        pltpu.sync_copy(zeros_h.at[pl.ds(0, _WIN)], tmp)

        @pl.when(s < 15)
        def _():
            pltpu.sync_copy(tmp.at[pl.ds(0, _WIN)],
                            acc_sh.at[pl.ds(s * _SL, _WIN)])
            pltpu.sync_copy(tmp.at[pl.ds(0, _SL - _WIN)],
                            acc_sh.at[pl.ds(s * _SL + _WIN, _SL - _WIN)])

        @pl.when(s == 15)
        def _():
            pltpu.sync_copy(tmp.at[pl.ds(0, _WIN)],
                            acc_sh.at[pl.ds(15 * _SL, _WIN)])
            pltpu.sync_copy(tmp.at[pl.ds(0, _SL_LAST - _WIN)],
                            acc_sh.at[pl.ds(15 * _SL + _WIN, _SL_LAST - _WIN)])

        @pl.when(rnd == 0)
        def _():
            pltpu.sync_copy(xf, repl)

        @pl.when(rnd > 0)
        def _():
            pltpu.sync_copy(outA, repl)
            for win in range(_N // _WIN):
                pltpu.sync_copy(outB.at[pl.ds(win * _WIN, _WIN)], tmp)

                def _addw(t, carry2):
                    off = t * 80
                    for u in range(5):
                        o = off + u * 16
                        repl[pl.ds(win * _WIN + o, 16)] = (
                            repl[pl.ds(win * _WIN + o, 16)]
                            + tmp[pl.ds(o, 16)])
                    return carry2
                lax.fori_loop(0, _WIN // 80, _addw, 0)

            @pl.when(wid < _N // _CURW)
            def _():
                cbase = wid * _CURW

                @pl.when(rnd == 1)
                def _():
                    pltpu.sync_copy(repl.at[pl.ds(cbase, _CURW)],
                                    cur1.at[pl.ds(cbase, _CURW)])

                @pl.when(rnd == 2)
                def _():
                    pltpu.sync_copy(repl.at[pl.ds(cbase, _CURW)],
                                    cur2.at[pl.ds(cbase, _CURW)])

                @pl.when(rnd == 3)
                def _():
                    pltpu.sync_copy(repl.at[pl.ds(cbase, _CURW)],
                                    cur3.at[pl.ds(cbase, _CURW)])

        @pl.when(rnd > 0)
        def _():
            pl.semaphore_signal(rsem, 1, core_index=1 - c)
            pl.semaphore_wait(rsem, 1)

        plsc.subcore_barrier()

        _body(0, 0, False, True)
        _body(1, 1, False, True)

        def _triple(i, carry2):
            k = 3 * i + 2
            _body(k, 2, True, True)
            _body(k + 1, 0, True, True)
            _body(k + 2, 1, True, True)
            return carry2
        lax.fori_loop(0, (nch - 4) // 3, _triple, 0)

        _body(nch - 2, 2, True, True)
        _body(nch - 1, 0, True, False)
        _drain1()
        _drain1()

        plsc.subcore_barrier()

        @pl.when(c == 0)
        def _():
            _acc_to(outA)

        @pl.when(c == 1)
        def _():
            _acc_to(outB)

        @pl.when(rnd < 3)
        def _():
            plsc.subcore_barrier()
            pl.semaphore_signal(rsem, 1, core_index=1 - c)
            pl.semaphore_wait(rsem, 1)

        return carry

    lax.fori_loop(0, 4, _round, 0)


_spmv4 = pl.kernel(
    _spmv4_body,
    out_type=(jax.ShapeDtypeStruct((_N,), jnp.float32),
              jax.ShapeDtypeStruct((_N,), jnp.float32),
              jax.ShapeDtypeStruct((_N,), jnp.float32),
              jax.ShapeDtypeStruct((_N,), jnp.float32),
              jax.ShapeDtypeStruct((_N,), jnp.float32)),
    mesh=plsc.VectorSubcoreMesh(core_axis_name="c", subcore_axis_name="s"),
    scratch_types=[
        pltpu.VMEM((_N,), jnp.float32),
        pltpu.VMEM((_WIN,), jnp.float32),
        pltpu.VMEM((3, _CH, _ROWW), jnp.int32),
        pltpu.VMEM((3, _CH, _ROWW), jnp.int32),
        pltpu.VMEM((3, _CH, _ROWW), jnp.float32),
        pltpu.VMEM((3, _CH, _ROWW), jnp.float32),
        pltpu.VMEM_SHARED((_N,), jnp.float32),
        pltpu.SemaphoreType.DMA,
        pltpu.SemaphoreType.DMA,
        pltpu.SemaphoreType.REGULAR,
    ],
    compiler_params=pltpu.CompilerParams(needs_layout_passes=False),
)


_B = 100
_BB = 10
_ROWS_BLK = _BB * 1000


def _stats_body(f5, x2, h5, h0, b2, mean_out, inv_out, acc1, acc2):
    i = pl.program_id(0)
    y = (jax.lax.dot_general(x2[...], h0[...], (((1,), (0,)), ((), ())),
                             preferred_element_type=jnp.float32)
         + jax.lax.dot_general(f5[...], h5[...], (((1,), (0,)), ((), ())),
                               preferred_element_type=jnp.float32)
         + b2[...])
    s1 = jnp.zeros((1000, 64), jnp.float32)
    s2 = jnp.zeros((1000, 64), jnp.float32)
    for b in range(_BB):
        yb = y[b * 1000:(b + 1) * 1000, :]
        s1 = s1 + yb
        s2 = s2 + yb * yb

    @pl.when(i == 0)
    def _():
        acc1[...] = s1
        acc2[...] = s2

    @pl.when(i > 0)
    def _():
        acc1[...] = acc1[...] + s1
        acc2[...] = acc2[...] + s2

    @pl.when(i == _B // _BB - 1)
    def _():
        denom = float(_B * 64)
        m = jnp.sum(acc1[...], axis=1, keepdims=True) / denom
        ey2 = jnp.sum(acc2[...], axis=1, keepdims=True) / denom
        var = ey2 - m * m
        mean_out[...] = m
        inv_out[...] = jax.lax.rsqrt(var + 1e-5)


def _norm_body(f5, x2, h5, h0, b2, g2, be2, mean, inv, out):
    y = (jax.lax.dot_general(x2[...], h0[...], (((1,), (0,)), ((), ())),
                             preferred_element_type=jnp.float32)
         + jax.lax.dot_general(f5[...], h5[...], (((1,), (0,)), ((), ())),
                               preferred_element_type=jnp.float32)
         + b2[...])
    scale = inv[...] * g2[...]
    shift = be2[...] - mean[...] * scale
    scale_r = jnp.concatenate([scale] * _BB, axis=0)
    shift_r = jnp.concatenate([shift] * _BB, axis=0)
    res = y * scale_r + shift_r
    out[...] = res.reshape(_BB, 1000, 64)


def kernel(x, edge_index, edge_weights, weight, bias, bn_gamma, bn_beta):
    src_p = edge_index[0].reshape(_ROWS, _ROWW)
    dst_p = edge_index[1].reshape(_ROWS, _ROWW)
    w_p = edge_weights.reshape(_ROWS, _ROWW)
    xf = x.reshape(_N)
    zero_n = jnp.zeros((_N,), jnp.float32)

    a4A, a4B, c1, c2, c3 = _spmv4(xf, zero_n, src_p, dst_p, w_p)

    f5 = jnp.stack([c1, c2, c3, a4A, a4B], axis=1)
    h = jnp.transpose(weight.reshape(64, 5), (1, 0))
    h5 = jnp.concatenate([h[1:4], h[4:5], h[4:5]], axis=0)
    h0 = h[0:1]
    b2 = bias.reshape(1, 64)
    g2 = bn_gamma.reshape(1000, 1)
    be2 = bn_beta.reshape(1000, 1)
    x2 = x.reshape(_N, 1)

    grid = (_B // _BB,)
    mean, inv = pl.pallas_call(
        _stats_body,
        grid=grid,
        in_specs=[
            pl.BlockSpec((_ROWS_BLK, 5), lambda i: (i, 0)),
            pl.BlockSpec((_ROWS_BLK, 1), lambda i: (i, 0)),
            pl.BlockSpec((5, 64), lambda i: (0, 0)),
            pl.BlockSpec((1, 64), lambda i: (0, 0)),
            pl.BlockSpec((1, 64), lambda i: (0, 0)),
        ],
        out_specs=[
            pl.BlockSpec((1000, 1), lambda i: (0, 0)),
            pl.BlockSpec((1000, 1), lambda i: (0, 0)),
        ],
        out_shape=[
            jax.ShapeDtypeStruct((1000, 1), jnp.float32),
            jax.ShapeDtypeStruct((1000, 1), jnp.float32),
        ],
        scratch_shapes=[
            pltpu.VMEM((1000, 64), jnp.float32),
            pltpu.VMEM((1000, 64), jnp.float32),
        ],
    )(f5, x2, h5, h0, b2)

    out = pl.pallas_call(
        _norm_body,
        grid=grid,
        in_specs=[
            pl.BlockSpec((_ROWS_BLK, 5), lambda i: (i, 0)),
            pl.BlockSpec((_ROWS_BLK, 1), lambda i: (i, 0)),
            pl.BlockSpec((5, 64), lambda i: (0, 0)),
            pl.BlockSpec((1, 64), lambda i: (0, 0)),
            pl.BlockSpec((1, 64), lambda i: (0, 0)),
            pl.BlockSpec((1000, 1), lambda i: (0, 0)),
            pl.BlockSpec((1000, 1), lambda i: (0, 0)),
            pl.BlockSpec((1000, 1), lambda i: (0, 0)),
            pl.BlockSpec((1000, 1), lambda i: (0, 0)),
        ],
        out_specs=pl.BlockSpec((_BB, 1000, 64), lambda i: (i, 0, 0)),
        out_shape=jax.ShapeDtypeStruct((_B, 1000, 64), jnp.float32),
    )(f5, x2, h5, h0, b2, g2, be2, mean, inv)
    return out


# speedup vs baseline: 212.6561x; 1.0136x over previous
import jax
import jax.numpy as jnp
from jax import lax
from jax.experimental import pallas as pl
from jax.experimental.pallas import tpu as pltpu
from jax.experimental.pallas import tpu_sc as plsc

_N = 100000
_E = 3200000
_ROWW = 128
_ROWS = _E // _ROWW
_CH = 8
_CBIG, _CSML = 100, 97
_NBIG = 7
_WIN = 4000
_SL = 6256
_SL_LAST = _N - 15 * _SL
_CURW = 4000


def _spmv4_body(xf, zeros_h, src2d, dst2d, w2d,
                outA, outB, cur1, cur2, cur3,
                repl, tmp, tmp2, sbuf, dbuf, wbuf, vbuf, acc_sh,
                sem_in, sem_sc, sem_cb, rsem):
    c = lax.axis_index("c")
    s = lax.axis_index("s")
    wid = c * 16 + s

    nch = jnp.where(wid < _NBIG, _CBIG, _CSML)
    row0 = jnp.where(wid < _NBIG, wid * (_CBIG * _CH),
                     _NBIG * (_CBIG * _CH) + (wid - _NBIG) * (_CSML * _CH))

    def _stage(k, set_):
        rbase = row0 + k * _CH
        pltpu.async_copy(src2d.at[pl.ds(rbase, _CH)], sbuf.at[set_], sem_in)
        pltpu.async_copy(dst2d.at[pl.ds(rbase, _CH)], dbuf.at[set_], sem_in)
        pltpu.async_copy(w2d.at[pl.ds(rbase, _CH)], wbuf.at[set_], sem_in)

    def _drain1():
        pltpu.make_async_copy(w2d.at[pl.ds(0, _CH)], vbuf.at[0],
                              sem_sc).wait()

    def _wait_stage(set_):
        h = pltpu.make_async_copy(src2d.at[pl.ds(0, _CH)], sbuf.at[set_],
                                  sem_in)
        h.wait()
        h.wait()
        h.wait()

    def _body(k, set_, drain, stage_next):
        if drain:
            _drain1()
        if stage_next:
            _stage(k + 1, (set_ + 1) % 3)
        _wait_stage(set_)
        for r in range(_CH):
            for j in range(_ROWW // 16):
                idx = sbuf[set_, r, pl.ds(j * 16, 16)]
                vals = plsc.load_gather(repl, [idx])
                vbuf[set_, r, pl.ds(j * 16, 16)] = (
                    vals * wbuf[set_, r, pl.ds(j * 16, 16)])
            pltpu.async_copy(vbuf.at[set_, r], acc_sh.at[dbuf.at[set_, r]],
                             sem_sc, add=True)

    def _acc_to(dst):
        @pl.when(s < 15)
        def _():
            base = s * _SL
            pltpu.sync_copy(acc_sh.at[pl.ds(base, _WIN)],
                            tmp.at[pl.ds(0, _WIN)])
            pltpu.sync_copy(tmp.at[pl.ds(0, _WIN)],
                            dst.at[pl.ds(base, _WIN)])
            pltpu.sync_copy(acc_sh.at[pl.ds(base + _WIN, _SL - _WIN)],
                            tmp.at[pl.ds(0, _SL - _WIN)])
            pltpu.sync_copy(tmp.at[pl.ds(0, _SL - _WIN)],
                            dst.at[pl.ds(base + _WIN, _SL - _WIN)])

        @pl.when(s == 15)
        def _():
            base = 15 * _SL
            pltpu.sync_copy(acc_sh.at[pl.ds(base, _WIN)],
                            tmp.at[pl.ds(0, _WIN)])
            pltpu.sync_copy(tmp.at[pl.ds(0, _WIN)],
                            dst.at[pl.ds(base, _WIN)])
            pltpu.sync_copy(acc_sh.at[pl.ds(base + _WIN, _SL_LAST - _WIN)],
                            tmp.at[pl.ds(0, _SL_LAST - _WIN)])
            pltpu.sync_copy(tmp.at[pl.ds(0, _SL_LAST - _WIN)],
                            dst.at[pl.ds(base + _WIN, _SL_LAST - _WIN)])

    def _round(rnd, carry):
        _stage(0, 0)

        pltpu.sync_copy(zeros_h.at[pl.ds(0, _WIN)], tmp)

        @pl.when(s < 15)
        def _():
            pltpu.sync_copy(tmp.at[pl.ds(0, _WIN)],
                            acc_sh.at[pl.ds(s * _SL, _WIN)])
            pltpu.sync_copy(tmp.at[pl.ds(0, _SL - _WIN)],
                            acc_sh.at[pl.ds(s * _SL + _WIN, _SL - _WIN)])

        @pl.when(s == 15)
        def _():
            pltpu.sync_copy(tmp.at[pl.ds(0, _WIN)],
                            acc_sh.at[pl.ds(15 * _SL, _WIN)])
            pltpu.sync_copy(tmp.at[pl.ds(0, _SL_LAST - _WIN)],
                            acc_sh.at[pl.ds(15 * _SL + _WIN, _SL_LAST - _WIN)])

        @pl.when(rnd == 0)
        def _():
            pltpu.sync_copy(xf, repl)

        @pl.when(rnd > 0)
        def _():
            pltpu.sync_copy(outA, repl)

            def _wait_win():
                pltpu.make_async_copy(outB.at[pl.ds(0, _WIN)], tmp,
                                      sem_cb).wait()

            def _addwin(win, buf):
                def _addw(t, carry2):
                    off = t * 80
                    for u in range(5):
                        o = off + u * 16
                        repl[pl.ds(win * _WIN + o, 16)] = (
                            repl[pl.ds(win * _WIN + o, 16)]
                            + buf[pl.ds(o, 16)])
                    return carry2
                lax.fori_loop(0, _WIN // 80, _addw, 0)

            pltpu.async_copy(outB.at[pl.ds(0, _WIN)], tmp, sem_cb)
            pltpu.async_copy(outB.at[pl.ds(_WIN, _WIN)], tmp2, sem_cb)

            def _pairwin(i, carry2):
                w0 = 2 * i
                _wait_win()
                _addwin(w0, tmp)
                pltpu.async_copy(outB.at[pl.ds((w0 + 2) * _WIN, _WIN)], tmp,
                                 sem_cb)
                _wait_win()
                _addwin(w0 + 1, tmp2)

                @pl.when(i < (_N // _WIN) // 2 - 1)
                def _():
                    w3 = jnp.minimum(w0 + 3, _N // _WIN - 1)
                    pltpu.async_copy(outB.at[pl.ds(w3 * _WIN, _WIN)], tmp2,
                                     sem_cb)
                return carry2
            lax.fori_loop(0, (_N // _WIN) // 2, _pairwin, 0)
            _wait_win()
            _addwin(_N // _WIN - 1, tmp)

            @pl.when(wid < _N // _CURW)
            def _():
                cbase = wid * _CURW

                @pl.when(rnd == 1)
                def _():
                    pltpu.sync_copy(repl.at[pl.ds(cbase, _CURW)],
                                    cur1.at[pl.ds(cbase, _CURW)])

                @pl.when(rnd == 2)
                def _():
                    pltpu.sync_copy(repl.at[pl.ds(cbase, _CURW)],
                                    cur2.at[pl.ds(cbase, _CURW)])

                @pl.when(rnd == 3)
                def _():
                    pltpu.sync_copy(repl.at[pl.ds(cbase, _CURW)],
                                    cur3.at[pl.ds(cbase, _CURW)])

        @pl.when(rnd > 0)
        def _():
            pl.semaphore_signal(rsem, 1, core_index=1 - c)
            pl.semaphore_wait(rsem, 1)

        plsc.subcore_barrier()

        _body(0, 0, False, True)
        _body(1, 1, False, True)

        def _triple(i, carry2):
            k = 3 * i + 2
            _body(k, 2, True, True)
            _body(k + 1, 0, True, True)
            _body(k + 2, 1, True, True)
            return carry2
        lax.fori_loop(0, (nch - 4) // 3, _triple, 0)

        _body(nch - 2, 2, True, True)
        _body(nch - 1, 0, True, False)
        _drain1()
        _drain1()

        plsc.subcore_barrier()

        @pl.when(c == 0)
        def _():
            _acc_to(outA)

        @pl.when(c == 1)
        def _():
            _acc_to(outB)

        @pl.when(rnd < 3)
        def _():
            plsc.subcore_barrier()
            pl.semaphore_signal(rsem, 1, core_index=1 - c)
            pl.semaphore_wait(rsem, 1)

        return carry

    lax.fori_loop(0, 4, _round, 0)


_spmv4 = pl.kernel(
    _spmv4_body,
    out_type=(jax.ShapeDtypeStruct((_N,), jnp.float32),
              jax.ShapeDtypeStruct((_N,), jnp.float32),
              jax.ShapeDtypeStruct((_N,), jnp.float32),
              jax.ShapeDtypeStruct((_N,), jnp.float32),
              jax.ShapeDtypeStruct((_N,), jnp.float32)),
    mesh=plsc.VectorSubcoreMesh(core_axis_name="c", subcore_axis_name="s"),
    scratch_types=[
        pltpu.VMEM((_N,), jnp.float32),
        pltpu.VMEM((_WIN,), jnp.float32),
        pltpu.VMEM((_WIN,), jnp.float32),
        pltpu.VMEM((3, _CH, _ROWW), jnp.int32),
        pltpu.VMEM((3, _CH, _ROWW), jnp.int32),
        pltpu.VMEM((3, _CH, _ROWW), jnp.float32),
        pltpu.VMEM((3, _CH, _ROWW), jnp.float32),
        pltpu.VMEM_SHARED((_N,), jnp.float32),
        pltpu.SemaphoreType.DMA,
        pltpu.SemaphoreType.DMA,
        pltpu.SemaphoreType.DMA,
        pltpu.SemaphoreType.REGULAR,
    ],
    compiler_params=pltpu.CompilerParams(needs_layout_passes=False),
)


_B = 100
_BB = 10
_ROWS_BLK = _BB * 1000


def _stats_body(f5, x2, h5, h0, b2, mean_out, inv_out, acc1, acc2):
    i = pl.program_id(0)
    y = (jax.lax.dot_general(x2[...], h0[...], (((1,), (0,)), ((), ())),
                             preferred_element_type=jnp.float32)
         + jax.lax.dot_general(f5[...], h5[...], (((1,), (0,)), ((), ())),
                               preferred_element_type=jnp.float32)
         + b2[...])
    s1 = jnp.zeros((1000, 64), jnp.float32)
    s2 = jnp.zeros((1000, 64), jnp.float32)
    for b in range(_BB):
        yb = y[b * 1000:(b + 1) * 1000, :]
        s1 = s1 + yb
        s2 = s2 + yb * yb

    @pl.when(i == 0)
    def _():
        acc1[...] = s1
        acc2[...] = s2

    @pl.when(i > 0)
    def _():
        acc1[...] = acc1[...] + s1
        acc2[...] = acc2[...] + s2

    @pl.when(i == _B // _BB - 1)
    def _():
        denom = float(_B * 64)
        m = jnp.sum(acc1[...], axis=1, keepdims=True) / denom
        ey2 = jnp.sum(acc2[...], axis=1, keepdims=True) / denom
        var = ey2 - m * m
        mean_out[...] = m
        inv_out[...] = jax.lax.rsqrt(var + 1e-5)


def _norm_body(f5, x2, h5, h0, b2, g2, be2, mean, inv, out):
    y = (jax.lax.dot_general(x2[...], h0[...], (((1,), (0,)), ((), ())),
                             preferred_element_type=jnp.float32)
         + jax.lax.dot_general(f5[...], h5[...], (((1,), (0,)), ((), ())),
                               preferred_element_type=jnp.float32)
         + b2[...])
    scale = inv[...] * g2[...]
    shift = be2[...] - mean[...] * scale
    scale_r = jnp.concatenate([scale] * _BB, axis=0)
    shift_r = jnp.concatenate([shift] * _BB, axis=0)
    res = y * scale_r + shift_r
    out[...] = res.reshape(_BB, 1000, 64)


def kernel(x, edge_index, edge_weights, weight, bias, bn_gamma, bn_beta):
    src_p = edge_index[0].reshape(_ROWS, _ROWW)
    dst_p = edge_index[1].reshape(_ROWS, _ROWW)
    w_p = edge_weights.reshape(_ROWS, _ROWW)
    xf = x.reshape(_N)
    zero_n = jnp.zeros((_N,), jnp.float32)

    a4A, a4B, c1, c2, c3 = _spmv4(xf, zero_n, src_p, dst_p, w_p)

    f5 = jnp.stack([c1, c2, c3, a4A, a4B], axis=1)
    h = jnp.transpose(weight.reshape(64, 5), (1, 0))
    h5 = jnp.concatenate([h[1:4], h[4:5], h[4:5]], axis=0)
    h0 = h[0:1]
    b2 = bias.reshape(1, 64)
    g2 = bn_gamma.reshape(1000, 1)
    be2 = bn_beta.reshape(1000, 1)
    x2 = x.reshape(_N, 1)

    grid = (_B // _BB,)
    mean, inv = pl.pallas_call(
        _stats_body,
        grid=grid,
        in_specs=[
            pl.BlockSpec((_ROWS_BLK, 5), lambda i: (i, 0)),
            pl.BlockSpec((_ROWS_BLK, 1), lambda i: (i, 0)),
            pl.BlockSpec((5, 64), lambda i: (0, 0)),
            pl.BlockSpec((1, 64), lambda i: (0, 0)),
            pl.BlockSpec((1, 64), lambda i: (0, 0)),
        ],
        out_specs=[
            pl.BlockSpec((1000, 1), lambda i: (0, 0)),
            pl.BlockSpec((1000, 1), lambda i: (0, 0)),
        ],
        out_shape=[
            jax.ShapeDtypeStruct((1000, 1), jnp.float32),
            jax.ShapeDtypeStruct((1000, 1), jnp.float32),
        ],
        scratch_shapes=[
            pltpu.VMEM((1000, 64), jnp.float32),
            pltpu.VMEM((1000, 64), jnp.float32),
        ],
    )(f5, x2, h5, h0, b2)

    out = pl.pallas_call(
        _norm_body,
        grid=grid,
        in_specs=[
            pl.BlockSpec((_ROWS_BLK, 5), lambda i: (i, 0)),
            pl.BlockSpec((_ROWS_BLK, 1), lambda i: (i, 0)),
            pl.BlockSpec((5, 64), lambda i: (0, 0)),
            pl.BlockSpec((1, 64), lambda i: (0, 0)),
            pl.BlockSpec((1, 64), lambda i: (0, 0)),
            pl.BlockSpec((1000, 1), lambda i: (0, 0)),
            pl.BlockSpec((1000, 1), lambda i: (0, 0)),
            pl.BlockSpec((1000, 1), lambda i: (0, 0)),
            pl.BlockSpec((1000, 1), lambda i: (0, 0)),
        ],
        out_specs=pl.BlockSpec((_BB, 1000, 64), lambda i: (i, 0, 0)),
        out_shape=jax.ShapeDtypeStruct((_B, 1000, 64), jnp.float32),
    )(f5, x2, h5, h0, b2, g2, be2, mean, inv)
    return out
